# Initial kernel scaffold; baseline (speedup 1.0000x reference)
import functools

import jax
import jax.numpy as jnp
from jax import lax
from jax.experimental import pallas as pl
from jax.experimental.pallas import tpu as pltpu
from jax.experimental.pallas import tpu_sc as plsc

N = 50000
H = 4
NPAD = 50176
ROWS_PER_SUB = NPAD // 16
ZCHUNK = 784
K = 1024
EP = 16 * 52 * K
NCHUNK = EP // K
f32 = jnp.float32
i32 = jnp.int32

_mesh = plsc.VectorSubcoreMesh(core_axis_name="c", subcore_axis_name="s")


def _zero_rows(buf, ncols):
    @pl.loop(0, ZCHUNK)
    def _(r):
        for sub in range(ncols // 16):
            buf[r, pl.ds(16 * sub, 16)] = jnp.zeros((16,), f32)


def _zero_slab(slab, buf, sid):
    for t in range(ROWS_PER_SUB // ZCHUNK):
        pltpu.sync_copy(buf.at[pl.ds(0, ZCHUNK)],
                        slab.at[pl.ds(sid * ROWS_PER_SUB + t * ZCHUNK, ZCHUNK)])


def _make_e_kernel():
    cps = NCHUNK // 2 // 16

    @functools.partial(
        pl.kernel, mesh=_mesh,
        out_type=[jax.ShapeDtypeStruct((4 * EP,), f32),
                  jax.ShapeDtypeStruct((2 * NPAD, 16), f32)],
        scratch_types=[
            pltpu.VMEM((8, 128), i32),
            pltpu.VMEM((8, 128), i32),
            pltpu.VMEM((K, 16), f32),
            pltpu.VMEM((K, 16), f32),
            pltpu.VMEM((K, 16), f32),
            pltpu.VMEM((4 * K,), f32),
            pltpu.VMEM((16,), f32),
            pltpu.VMEM_SHARED((NPAD, 16), f32),
        ])
    def ekern(als_hbm, ald_hbm, src_hbm, dst_hbm, stab_hbm,
              e_out, ssum_out, srcv, dstv, arows, brows, erows, ev, stabv, slab):
        cid = lax.axis_index("c")
        sid = lax.axis_index("s")
        pltpu.sync_copy(stab_hbm.at[0], stabv)
        lane = lax.iota(i32, 16)
        lmask = lane < 4
        rowsel = lane >> 2
        colsel = lane & 3

        _zero_rows(erows, 16)
        _zero_slab(slab, erows, sid)
        plsc.subcore_barrier()

        base_chunk = cid * (NCHUNK // 2) + sid * cps

        @pl.loop(0, cps)
        def _(t):
            chunk = base_chunk + t
            pltpu.sync_copy(src_hbm.at[pl.ds(8 * chunk, 8)], srcv)
            pltpu.sync_copy(dst_hbm.at[pl.ds(8 * chunk, 8)], dstv)
            for q in range(8):
                pltpu.sync_copy(als_hbm.at[srcv.at[q]],
                                arows.at[pl.ds(128 * q, 128)])
                pltpu.sync_copy(ald_hbm.at[dstv.at[q]],
                                brows.at[pl.ds(128 * q, 128)])
            stab = stabv[...]

            @pl.loop(0, K)
            def _(r):
                al = arows[r, pl.ds(0, 16)] + brows[r, pl.ds(0, 16)]
                al = jnp.where(al > 0, al, al * 0.2) - stab
                e16 = jnp.where(lmask, jnp.exp(al), 0.0)
                erows[r, pl.ds(0, 16)] = e16

            @pl.loop(0, K // 4)
            def _(j):
                ridx = 4 * j + rowsel
                ev[pl.ds(16 * j, 16)] = plsc.load_gather(erows, [ridx, colsel])

            for q in range(8):
                pltpu.sync_copy(erows.at[pl.ds(128 * q, 128)],
                                slab.at[dstv.at[q]], add=True)
            pltpu.sync_copy(ev, e_out.at[pl.ds(4 * K * chunk, 4 * K)])

        plsc.subcore_barrier()
        pltpu.sync_copy(
            slab.at[pl.ds(sid * ROWS_PER_SUB, ROWS_PER_SUB)],
            ssum_out.at[pl.ds(cid * NPAD + sid * ROWS_PER_SUB, ROWS_PER_SUB)])

    return ekern


def _make_msg_kernel(NG, C):
    hpg = 2 if C <= 16 else 1
    CG = hpg * C
    cps = NCHUNK // 16

    @functools.partial(
        pl.kernel, mesh=_mesh,
        out_type=jax.ShapeDtypeStruct((NG * NPAD, CG), f32),
        scratch_types=[
            pltpu.VMEM((8, 128), i32),
            pltpu.VMEM((8, 128), i32),
            pltpu.VMEM((4 * K,), f32),
            pltpu.VMEM((K, CG), f32),
            pltpu.VMEM_SHARED((NPAD, CG), f32),
        ])
    def mkern(h_hbm, src_hbm, dst_hbm, e_hbm,
              out_hbm, srcv, dstv, ev, rows, slab):
        cid = lax.axis_index("c")
        sid = lax.axis_index("s")
        lane = lax.iota(i32, 16)

        for gi in range(NG // 2):
            g = cid * (NG // 2) + gi
            goff = g * NPAD
            hb = g * hpg
            _zero_rows(rows, CG)
            _zero_slab(slab, rows, sid)
            plsc.subcore_barrier()

            @pl.loop(0, cps)
            def _(t):
                chunk = sid * cps + t
                pltpu.sync_copy(src_hbm.at[pl.ds(8 * chunk, 8)], srcv)
                pltpu.sync_copy(dst_hbm.at[pl.ds(8 * chunk, 8)], dstv)
                pltpu.sync_copy(e_hbm.at[pl.ds(4 * K * chunk, 4 * K)], ev)
                for q in range(8):
                    for v in range(8):
                        srcv[q, pl.ds(16 * v, 16)] = (
                            srcv[q, pl.ds(16 * v, 16)] + goff)
                for q in range(8):
                    pltpu.sync_copy(h_hbm.at[srcv.at[q]],
                                    rows.at[pl.ds(128 * q, 128)])

                if C >= 16:
                    @pl.loop(0, K)
                    def _(r):
                        for sub in range(CG // 16):
                            w = ev[4 * r + hb + (sub * 16) // C]
                            rows[r, pl.ds(16 * sub, 16)] = (
                                rows[r, pl.ds(16 * sub, 16)] * w)
                else:
                    @pl.loop(0, K)
                    def _(r):
                        w0 = ev[4 * r + hb]
                        w1 = ev[4 * r + hb + 1]
                        wv = jnp.where(lane < 8, w0, w1)
                        rows[r, pl.ds(0, 16)] = rows[r, pl.ds(0, 16)] * wv

                for q in range(8):
                    pltpu.sync_copy(rows.at[pl.ds(128 * q, 128)],
                                    slab.at[dstv.at[q]], add=True)

            plsc.subcore_barrier()
            pltpu.sync_copy(
                slab.at[pl.ds(sid * ROWS_PER_SUB, ROWS_PER_SUB)],
                out_hbm.at[pl.ds(goff + sid * ROWS_PER_SUB, ROWS_PER_SUB)])
            plsc.subcore_barrier()

    return mkern


_e_kernel = _make_e_kernel()
_msg_kernels = {C: _make_msg_kernel(NG, C) for C, NG in ((16, 2), (32, 4), (8, 2))}


def _bn_dense(x, g, b, eps=1e-5):
    m = x.mean(axis=0)
    v = x.var(axis=0)
    return g * (x - m) / jnp.sqrt(v + eps) + b


def _pad_rows(a):
    return jnp.pad(a, ((0, NPAD - a.shape[0]),) + ((0, 0),) * (a.ndim - 1))


def _gat_layer(xb, src2d, dst2d, W, a_s, a_d, bias):
    C = a_s.shape[1]
    hpg = 2 if C <= 16 else 1
    NG = H // hpg
    CG = hpg * C
    dout = H * C

    h = xb @ W
    h3 = h.reshape(N, H, C)
    als = (h3 * a_s[None]).sum(-1)
    ald = (h3 * a_d[None]).sum(-1)
    stab4 = als.max(0) + ald.max(0)
    stab16 = jnp.tile(stab4, 4).reshape(1, 16)
    als16 = _pad_rows(jnp.concatenate([als, jnp.zeros((N, 12), f32)], axis=1))
    ald16 = _pad_rows(jnp.concatenate([ald, jnp.zeros((N, 12), f32)], axis=1))

    e_arr, ssum = _e_kernel(als16, ald16, src2d, dst2d, stab16)

    hg = jnp.transpose(h3.reshape(N, NG, CG), (1, 0, 2))
    hg = jnp.pad(hg, ((0, 0), (0, NPAD - N), (0, 0))).reshape(NG * NPAD, CG)

    out_all = _msg_kernels[C](hg, src2d, dst2d, e_arr)

    ssum = ssum.reshape(2, NPAD, 16)
    ssum = ssum[0, :N, :4] + ssum[1, :N, :4]
    inv = 1.0 / (ssum + 1e-16)

    out3 = out_all.reshape(NG, NPAD, CG)[:, :N, :].reshape(NG, N, hpg, C)
    out3 = jnp.transpose(out3, (1, 0, 2, 3)).reshape(N, H, C)
    out3 = out3 * inv[:, :, None]
    return out3.reshape(N, dout) + bias


def kernel(x, edge_index, edge_attr, bn_g, bn_b,
           W1, as1, ad1, b1, g1, be1,
           W2, as2, ad2, b2, g2, be2,
           W3, as3, ad3, b3, g3, be3,
           W4, as4, ad4, b4, g4, be4, Wo, bo):
    sl = jnp.arange(N, dtype=i32)
    src = jnp.concatenate([edge_index[0].astype(i32), sl])
    dst = jnp.concatenate([edge_index[1].astype(i32), sl])
    pad = jnp.full((EP - (N + edge_index.shape[1]),), N, i32)
    src2d = jnp.concatenate([src, pad]).reshape(EP // 128, 128)
    dst2d = jnp.concatenate([dst, pad]).reshape(EP // 128, 128)

    hb = _bn_dense(x[:, 0:4], bn_g, bn_b)
    h = _gat_layer(hb, src2d, dst2d, W1, as1, ad1, b1)
    h = jax.nn.relu(_bn_dense(h, g1, be1))
    h = _gat_layer(h, src2d, dst2d, W2, as2, ad2, b2)
    h = jax.nn.relu(_bn_dense(h, g2, be2))
    h = _gat_layer(h, src2d, dst2d, W3, as3, ad3, b3)
    h = jax.nn.relu(_bn_dense(h, g3, be3))
    h = _gat_layer(h, src2d, dst2d, W4, as4, ad4, b4)
    h = jax.nn.relu(_bn_dense(h, g4, be4))
    return (h @ Wo + bo).squeeze(-1)


# speedup vs baseline: 44.5465x; 44.5465x over previous
import dataclasses
import functools

import jax
import jax.numpy as jnp
from jax import lax
from jax.experimental import pallas as pl
from jax.experimental.pallas import tpu as pltpu
from jax.experimental.pallas import tpu_sc as plsc

N = 50000
H = 4
NPAD = 50176
ROWS_PER_SUB = NPAD // 16
ZCHUNK = 196
EP = 16 * 52 * 1024
KE = 1024
KM = 256
f32 = jnp.float32
i32 = jnp.int32

_mesh = plsc.VectorSubcoreMesh(core_axis_name="c", subcore_axis_name="s")

_sc_params = pltpu.CompilerParams()
for _fld, _val in (("needs_layout_passes", False),
                   ("use_tc_tiling_on_sc", False)):
    if _fld in pltpu.CompilerParams.__dataclass_fields__:
        _sc_params = dataclasses.replace(_sc_params, **{_fld: _val})


def _zero_rows(buf, ncols):
    @pl.loop(0, ZCHUNK)
    def _(r):
        for sub in range(ncols // 16):
            buf[r, pl.ds(16 * sub, 16)] = jnp.zeros((16,), f32)


def _zero_slab(slab, buf, sid):
    for t in range(ROWS_PER_SUB // ZCHUNK):
        pltpu.sync_copy(buf.at[pl.ds(0, ZCHUNK)],
                        slab.at[pl.ds(sid * ROWS_PER_SUB + t * ZCHUNK, ZCHUNK)])


def _make_e_kernel():
    K = KE
    QN = K // 128
    NCHUNK = EP // K
    cps = NCHUNK // 2 // 16

    @functools.partial(
        pl.kernel, mesh=_mesh, compiler_params=_sc_params,
        out_type=[jax.ShapeDtypeStruct((4 * EP,), f32),
                  jax.ShapeDtypeStruct((2 * NPAD, 16), f32)],
        scratch_types=[
            pltpu.VMEM((QN, 128), i32),
            pltpu.VMEM((QN, 128), i32),
            pltpu.VMEM((K, 16), f32),
            pltpu.VMEM((K, 16), f32),
            pltpu.VMEM((K, 16), f32),
            pltpu.VMEM((4 * K,), f32),
            pltpu.VMEM((16,), f32),
            pltpu.VMEM_SHARED((NPAD, 16), f32),
        ])
    def ekern(als_hbm, ald_hbm, src_hbm, dst_hbm, stab_hbm,
              e_out, ssum_out, srcv, dstv, arows, brows, erows, ev, stabv, slab):
        cid = lax.axis_index("c")
        sid = lax.axis_index("s")
        pltpu.sync_copy(stab_hbm.at[0], stabv)
        lane = lax.iota(i32, 16)
        lmask = lane < 4
        rowsel = lane >> 2
        colsel = lane & 3

        _zero_rows(erows, 16)
        _zero_slab(slab, erows, sid)
        plsc.subcore_barrier()

        base_chunk = cid * (NCHUNK // 2) + sid * cps

        @pl.loop(0, cps)
        def _(t):
            chunk = base_chunk + t
            pltpu.sync_copy(src_hbm.at[pl.ds(QN * chunk, QN)], srcv)
            pltpu.sync_copy(dst_hbm.at[pl.ds(QN * chunk, QN)], dstv)
            for q in range(QN):
                pltpu.sync_copy(als_hbm.at[srcv.at[q]],
                                arows.at[pl.ds(128 * q, 128)])
                pltpu.sync_copy(ald_hbm.at[dstv.at[q]],
                                brows.at[pl.ds(128 * q, 128)])
            stab = stabv[...]

            @pl.loop(0, K)
            def _(r):
                al = arows[r, pl.ds(0, 16)] + brows[r, pl.ds(0, 16)]
                al = jnp.where(al > 0, al, al * 0.2) - stab
                e16 = jnp.where(lmask, jnp.exp(al), 0.0)
                erows[r, pl.ds(0, 16)] = e16

            @pl.loop(0, K // 4)
            def _(j):
                ridx = 4 * j + rowsel
                ev[pl.ds(16 * j, 16)] = plsc.load_gather(erows, [ridx, colsel])

            for q in range(QN):
                pltpu.sync_copy(erows.at[pl.ds(128 * q, 128)],
                                slab.at[dstv.at[q]], add=True)
            pltpu.sync_copy(ev, e_out.at[pl.ds(4 * K * chunk, 4 * K)])

        plsc.subcore_barrier()
        pltpu.sync_copy(
            slab.at[pl.ds(sid * ROWS_PER_SUB, ROWS_PER_SUB)],
            ssum_out.at[pl.ds(cid * NPAD + sid * ROWS_PER_SUB, ROWS_PER_SUB)])

    return ekern


def _make_msg_kernel(NG, C):
    hpg = 2 if C <= 16 else 1
    CG = hpg * C
    K = KM
    QN = K // 128
    NCHUNK = EP // K
    cps = NCHUNK // 16

    @functools.partial(
        pl.kernel, mesh=_mesh, compiler_params=_sc_params,
        out_type=jax.ShapeDtypeStruct((NG * NPAD, CG), f32),
        scratch_types=[
            pltpu.VMEM((QN, 128), i32),
            pltpu.VMEM((QN, 128), i32),
            pltpu.VMEM((4 * K,), f32),
            pltpu.VMEM((K, CG), f32),
            pltpu.VMEM_SHARED((NPAD, CG), f32),
        ])
    def mkern(h_hbm, src_hbm, dst_hbm, e_hbm,
              out_hbm, srcv, dstv, ev, rows, slab):
        cid = lax.axis_index("c")
        sid = lax.axis_index("s")
        lane = lax.iota(i32, 16)

        for gi in range(NG // 2):
            g = cid * (NG // 2) + gi
            goff = g * NPAD
            hb = g * hpg
            _zero_rows(rows, CG)
            _zero_slab(slab, rows, sid)
            plsc.subcore_barrier()

            @pl.loop(0, cps)
            def _(t):
                chunk = sid * cps + t
                pltpu.sync_copy(src_hbm.at[pl.ds(QN * chunk, QN)], srcv)
                pltpu.sync_copy(dst_hbm.at[pl.ds(QN * chunk, QN)], dstv)
                pltpu.sync_copy(e_hbm.at[pl.ds(4 * K * chunk, 4 * K)], ev)
                for q in range(QN):
                    for v in range(8):
                        srcv[q, pl.ds(16 * v, 16)] = (
                            srcv[q, pl.ds(16 * v, 16)] + goff)
                for q in range(QN):
                    pltpu.sync_copy(h_hbm.at[srcv.at[q]],
                                    rows.at[pl.ds(128 * q, 128)])

                if C >= 16:
                    @pl.loop(0, K)
                    def _(r):
                        for sub in range(CG // 16):
                            widx = jnp.full((16,), 4 * r + hb + (sub * 16) // C,
                                            i32)
                            wv = plsc.load_gather(ev, [widx])
                            rows[r, pl.ds(16 * sub, 16)] = (
                                rows[r, pl.ds(16 * sub, 16)] * wv)
                else:
                    @pl.loop(0, K)
                    def _(r):
                        widx = 4 * r + hb + jnp.where(lane < 8, 0, 1)
                        wv = plsc.load_gather(ev, [widx])
                        rows[r, pl.ds(0, 16)] = rows[r, pl.ds(0, 16)] * wv

                for q in range(QN):
                    pltpu.sync_copy(rows.at[pl.ds(128 * q, 128)],
                                    slab.at[dstv.at[q]], add=True)

            plsc.subcore_barrier()
            pltpu.sync_copy(
                slab.at[pl.ds(sid * ROWS_PER_SUB, ROWS_PER_SUB)],
                out_hbm.at[pl.ds(goff + sid * ROWS_PER_SUB, ROWS_PER_SUB)])
            plsc.subcore_barrier()

    return mkern


_e_kernel = _make_e_kernel()
_msg_kernels = {C: _make_msg_kernel(NG, C) for C, NG in ((16, 2), (32, 4), (8, 2))}


def _bn_dense(x, g, b, eps=1e-5):
    m = x.mean(axis=0)
    v = x.var(axis=0)
    return g * (x - m) / jnp.sqrt(v + eps) + b


def _pad_rows(a):
    return jnp.pad(a, ((0, NPAD - a.shape[0]),) + ((0, 0),) * (a.ndim - 1))


def _gat_layer(xb, src2d, dst2d, W, a_s, a_d, bias):
    C = a_s.shape[1]
    hpg = 2 if C <= 16 else 1
    NG = H // hpg
    CG = hpg * C
    dout = H * C

    h = xb @ W
    h3 = h.reshape(N, H, C)
    als = (h3 * a_s[None]).sum(-1)
    ald = (h3 * a_d[None]).sum(-1)
    stab4 = als.max(0) + ald.max(0)
    stab16 = jnp.tile(stab4, 4).reshape(1, 16)
    als16 = _pad_rows(jnp.concatenate([als, jnp.zeros((N, 12), f32)], axis=1))
    ald16 = _pad_rows(jnp.concatenate([ald, jnp.zeros((N, 12), f32)], axis=1))

    e_arr, ssum = _e_kernel(als16, ald16, src2d, dst2d, stab16)

    hg = jnp.transpose(h3.reshape(N, NG, CG), (1, 0, 2))
    hg = jnp.pad(hg, ((0, 0), (0, NPAD - N), (0, 0))).reshape(NG * NPAD, CG)

    out_all = _msg_kernels[C](hg, src2d, dst2d, e_arr)

    ssum = ssum.reshape(2, NPAD, 16)
    ssum = ssum[0, :N, :4] + ssum[1, :N, :4]
    inv = 1.0 / (ssum + 1e-16)

    out3 = out_all.reshape(NG, NPAD, CG)[:, :N, :].reshape(NG, N, hpg, C)
    out3 = jnp.transpose(out3, (1, 0, 2, 3)).reshape(N, H, C)
    out3 = out3 * inv[:, :, None]
    return out3.reshape(N, dout) + bias


def kernel(x, edge_index, edge_attr, bn_g, bn_b,
           W1, as1, ad1, b1, g1, be1,
           W2, as2, ad2, b2, g2, be2,
           W3, as3, ad3, b3, g3, be3,
           W4, as4, ad4, b4, g4, be4, Wo, bo):
    sl = jnp.arange(N, dtype=i32)
    src = jnp.concatenate([edge_index[0].astype(i32), sl])
    dst = jnp.concatenate([edge_index[1].astype(i32), sl])
    pad = jnp.full((EP - (N + edge_index.shape[1]),), N, i32)
    src2d = jnp.concatenate([src, pad]).reshape(EP // 128, 128)
    dst2d = jnp.concatenate([dst, pad]).reshape(EP // 128, 128)

    hb = _bn_dense(x[:, 0:4], bn_g, bn_b)
    h = _gat_layer(hb, src2d, dst2d, W1, as1, ad1, b1)
    h = jax.nn.relu(_bn_dense(h, g1, be1))
    h = _gat_layer(h, src2d, dst2d, W2, as2, ad2, b2)
    h = jax.nn.relu(_bn_dense(h, g2, be2))
    h = _gat_layer(h, src2d, dst2d, W3, as3, ad3, b3)
    h = jax.nn.relu(_bn_dense(h, g3, be3))
    h = _gat_layer(h, src2d, dst2d, W4, as4, ad4, b4)
    h = jax.nn.relu(_bn_dense(h, g4, be4))
    return (h @ Wo + bo).squeeze(-1)
